# baseline (speedup 1.0000x reference)
import functools

import jax
import jax.numpy as jnp
from jax import lax
from jax.experimental import pallas as pl
from jax.experimental.pallas import tpu as pltpu
from jax.experimental.pallas import tpu_sc as plsc

NC = 2
NS = 16
NW = NC * NS
L = 16
D = 128


_MLP_BLK = 512


@functools.lru_cache(maxsize=None)
def _mlp_call(n_out):
    grid = (pl.cdiv(n_out, _MLP_BLK),)

    def body(a_ref, b_ref, w0_ref, w1_ref, bias_ref, o_ref):
        acc = jnp.dot(a_ref[...], w0_ref[...], preferred_element_type=jnp.float32)
        acc = acc + jnp.dot(b_ref[...], w1_ref[...], preferred_element_type=jnp.float32)
        acc = acc + bias_ref[0:1, :]
        o_ref[...] = acc * jax.nn.sigmoid(acc)

    return pl.pallas_call(
        body,
        grid=grid,
        in_specs=[
            pl.BlockSpec((_MLP_BLK, D), lambda i: (i, 0)),
            pl.BlockSpec((_MLP_BLK, D), lambda i: (i, 0)),
            pl.BlockSpec((D, D), lambda i: (0, 0)),
            pl.BlockSpec((D, D), lambda i: (0, 0)),
            pl.BlockSpec((8, D), lambda i: (0, 0)),
        ],
        out_specs=pl.BlockSpec((_MLP_BLK, D), lambda i: (i, 0)),
        out_shape=jax.ShapeDtypeStruct((n_out, D), jnp.float32),
    )


def _mlp(a, b, W, bias, n_out):
    w0 = W[:D]
    w1 = W[D:]
    bias8 = jnp.broadcast_to(bias[None, :], (8, D))
    return _mlp_call(n_out)(a, b, w0, w1, bias8)


_GB = 256


@functools.lru_cache(maxsize=None)
def _gather_call(n_pad, n_h):
    rows_per_w = n_pad // NW
    n_chunks = rows_per_w // _GB
    mesh = plsc.VectorSubcoreMesh(
        core_axis_name="c", subcore_axis_name="s", num_cores=NC, num_subcores=NS)

    @functools.partial(
        pl.kernel,
        out_type=[jax.ShapeDtypeStruct((n_pad, D), jnp.float32),
                  jax.ShapeDtypeStruct((n_pad, D), jnp.float32)],
        mesh=mesh,
        scratch_types=[
            pltpu.VMEM((_GB,), jnp.int32),
            pltpu.VMEM((_GB, D), jnp.float32),
            pltpu.SemaphoreType.DMA,
        ],
    )
    def k(h_hbm, i0_hbm, i1_hbm, g0_hbm, g1_hbm, ibuf, rbuf, sem):
        w = lax.axis_index("s") * NC + lax.axis_index("c")

        def chunk(ci, _):
            base = w * rows_per_w + ci * _GB
            for ih, gh in ((i0_hbm, g0_hbm), (i1_hbm, g1_hbm)):
                pltpu.sync_copy(ih.at[pl.ds(base, _GB)], ibuf)
                cps = [
                    pltpu.async_copy(
                        h_hbm.at[ibuf.at[pl.ds(j * 128, 128)]],
                        rbuf.at[pl.ds(j * 128, 128)], sem)
                    for j in range(_GB // 128)
                ]
                for cp in cps:
                    cp.wait()
                pltpu.sync_copy(rbuf, gh.at[pl.ds(base, _GB)])
            return 0

        lax.fori_loop(0, n_chunks, chunk, 0)

    return k


def _gather2(h, idx0, idx1):
    n = idx0.shape[0]
    n_pad = ((n + NW * _GB - 1) // (NW * _GB)) * (NW * _GB)
    pad = jnp.zeros((n_pad - n,), jnp.int32)
    i0 = jnp.concatenate([idx0.astype(jnp.int32), pad])
    i1 = jnp.concatenate([idx1.astype(jnp.int32), pad])
    return _gather_call(n_pad, h.shape[0])(h, i0, i1)


def _scatter_add(h, c0, c1, m):
    return jnp.zeros((m, D), jnp.float32).at[c0].add(h).at[c1].add(h)


def kernel(feat, child2_0, child2_1, child3_0, child3_1, child4_0, child4_1,
           W_up2, b_up2, W_up3, b_up3, W_up4, b_up4,
           W_down3, b_down3, W_down2, b_down2, W_down1, b_down1):
    n1 = feat.shape[0]
    n2 = child2_0.shape[0]
    n3 = child3_0.shape[0]
    n4 = child4_0.shape[0]

    g0, g1 = _gather2(feat, child2_0, child2_1)
    h2 = _mlp(g0, g1, W_up2, b_up2, n2)
    g0, g1 = _gather2(h2, child3_0, child3_1)
    h3 = _mlp(g0, g1, W_up3, b_up3, n3)
    g0, g1 = _gather2(h3, child4_0, child4_1)
    h4 = _mlp(g0, g1, W_up4, b_up4, n4)

    hd3 = _scatter_add(h4, child4_0, child4_1, n3)
    h3 = _mlp(h3, hd3, W_down3, b_down3, n3)
    hd2 = _scatter_add(h3, child3_0, child3_1, n2)
    h2 = _mlp(h2, hd2, W_down2, b_down2, n2)
    hd1 = _scatter_add(h2, child2_0, child2_1, n1)
    h1 = _mlp(feat, hd1, W_down1, b_down1, n1)
    return h1
